# baseline (speedup 1.0000x reference)
import functools

import jax
import jax.numpy as jnp
from jax import lax
from jax.experimental import pallas as pl
from jax.experimental.pallas import tpu as pltpu
from jax.experimental.pallas import tpu_sc as plsc

VOCAB = 1000000
EMBED = 64
BATCH = 16384

NUM_CORES = 2
NUM_SUBCORES = 16
LANES = 16
NW = NUM_CORES * NUM_SUBCORES
B_PER_W = BATCH // NW
BLOCKS = B_PER_W // LANES

_LN2 = 0.6931471805599453
_SQRT2 = 1.4142135623730951


def _log_f32(s):
    i = plsc.bitcast(s, jnp.int32)
    e = (i >> 23) - 127
    m = plsc.bitcast((i & 0x007FFFFF) | 0x3F800000, jnp.float32)
    adj = m > _SQRT2
    m = jnp.where(adj, m * 0.5, m)
    e = (e + adj.astype(jnp.int32)).astype(jnp.float32)
    t = (m - 1.0) / (m + 1.0)
    t2 = t * t
    log_m = 2.0 * t * (1.0 + t2 * (1.0 / 3.0 + t2 * (0.2 + t2 * (1.0 / 7.0))))
    return e * _LN2 + log_m


def _log_softmax_inplace(rows_ref):
    def block_body(blk, _):
        iv = blk * LANES + lax.iota(jnp.int32, LANES)

        def col(c):
            ic = jnp.full((LANES,), c, jnp.int32)
            return ic, plsc.load_gather(rows_ref, [iv, ic])

        def max_body(c, mx):
            _, g = col(c)
            return jnp.maximum(mx, g)

        mx = lax.fori_loop(1, EMBED, max_body, col(0)[1], unroll=4)

        def sum_body(c, s):
            _, g = col(c)
            return s + jnp.exp(g - mx)

        s = lax.fori_loop(0, EMBED, sum_body, jnp.zeros((LANES,), jnp.float32),
                          unroll=4)
        lse = mx + _log_f32(s)

        def out_body(c, carry):
            ic, g = col(c)
            plsc.store_scatter(rows_ref, [iv, ic], g - lse)
            return carry

        lax.fori_loop(0, EMBED, out_body, 0, unroll=4)
        return _

    lax.fori_loop(0, BLOCKS, block_body, 0)


def _sc_body(tgt_hbm, ctx_hbm, ttab_hbm, ctab_hbm, out_t_hbm, out_c_hbm,
             idx_t, idx_c, rows_t, rows_c, sem_t, sem_c, sem_wt, sem_wc):
    wid = lax.axis_index("s") * NUM_CORES + lax.axis_index("c")
    base = wid * B_PER_W

    pltpu.sync_copy(tgt_hbm.at[pl.ds(base, B_PER_W)], idx_t)
    pltpu.sync_copy(ctx_hbm.at[pl.ds(base, B_PER_W)], idx_c)
    gather_t = pltpu.async_copy(ttab_hbm.at[idx_t], rows_t, sem_t)
    gather_c = pltpu.async_copy(ctab_hbm.at[idx_c], rows_c, sem_c)

    gather_t.wait()
    _log_softmax_inplace(rows_t)
    wb_t = pltpu.async_copy(rows_t, out_t_hbm.at[pl.ds(base, B_PER_W)], sem_wt)

    gather_c.wait()
    _log_softmax_inplace(rows_c)
    wb_c = pltpu.async_copy(rows_c, out_c_hbm.at[pl.ds(base, B_PER_W)], sem_wc)

    wb_t.wait()
    wb_c.wait()


@jax.jit
def _skipgram_sc(target, context, target_table, context_table):
    mesh = plsc.VectorSubcoreMesh(core_axis_name="c", subcore_axis_name="s")
    return pl.kernel(
        _sc_body,
        out_type=(
            jax.ShapeDtypeStruct((BATCH, EMBED), jnp.float32),
            jax.ShapeDtypeStruct((BATCH, EMBED), jnp.float32),
        ),
        mesh=mesh,
        compiler_params=pltpu.CompilerParams(
            needs_layout_passes=False, use_tc_tiling_on_sc=False),
        scratch_types=[
            pltpu.VMEM((B_PER_W,), jnp.int32),
            pltpu.VMEM((B_PER_W,), jnp.int32),
            pltpu.VMEM((B_PER_W, EMBED), jnp.float32),
            pltpu.VMEM((B_PER_W, EMBED), jnp.float32),
            pltpu.SemaphoreType.DMA,
            pltpu.SemaphoreType.DMA,
            pltpu.SemaphoreType.DMA,
            pltpu.SemaphoreType.DMA,
        ],
    )(target, context, target_table, context_table)


def kernel(target, context, target_table, context_table):
    return _skipgram_sc(target, context, target_table, context_table)
